# baseline (speedup 1.0000x reference)
import jax
import jax.numpy as jnp
from jax import lax
from jax.experimental import pallas as pl
from jax.experimental.pallas import tpu as pltpu
from jax.experimental.pallas import tpu_sc as plsc

_NC = 2
_NS = 16
_NW = _NC * _NS

_REP = 512


def _tc_add_body(geom_ref, tok_ref, out_ref):
    out_ref[...] = tok_ref[...] + geom_ref[...][:, None, :]


def _make_sc_geom(b, d, bpw):
    mesh = plsc.VectorSubcoreMesh(
        core_axis_name="c", subcore_axis_name="s",
        num_cores=_NC, num_subcores=_NS)

    def sc_geom(cid, ctable_rep):
        @pl.kernel(
            out_type=jax.ShapeDtypeStruct((b, d), jnp.float32),
            mesh=mesh,
            scratch_types=[
                pltpu.VMEM((bpw,), jnp.int32),
                pltpu.VMEM((bpw, d), jnp.float32),
                pltpu.SemaphoreType.DMA,
            ],
        )
        def run(cid_hbm, ctable_hbm, geom_hbm, c_v, rows_v, sem):
            wid = lax.axis_index("s") * _NC + lax.axis_index("c")
            base = wid * bpw
            pltpu.sync_copy(cid_hbm.at[pl.ds(base, bpw)], c_v)
            lane = lax.iota(jnp.int32, 16)
            for i in range(bpw // 16):
                s = pl.ds(i * 16, 16)
                rep = (base + i * 16 + lane) & (_REP - 1)
                c_v[s] = c_v[s] + rep * 4
            pltpu.async_copy(ctable_hbm.at[c_v], rows_v, sem).wait()
            pltpu.sync_copy(rows_v, geom_hbm.at[pl.ds(base, bpw)])

        return run(cid, ctable_rep)

    return sc_geom


def kernel(tokens, view_ids, side_ids, view_embed, side_embed):
    B, L, D = tokens.shape
    BB = 128
    NB = B // BB
    BPW = B // _NW

    cid = view_ids.astype(jnp.int32) * 2 + side_ids.astype(jnp.int32)

    ctable_rep = (view_embed[None, :, None, :]
                  + side_embed[None, None, :, :])
    ctable_rep = jnp.broadcast_to(ctable_rep, (_REP, 2, 2, D)).reshape(-1, D)

    geom = _make_sc_geom(B, D, BPW)(cid, ctable_rep)

    return pl.pallas_call(
        _tc_add_body,
        grid=(NB,),
        in_specs=[
            pl.BlockSpec((BB, D), lambda i: (i, 0)),
            pl.BlockSpec((BB, L, D), lambda i: (i, 0, 0)),
        ],
        out_specs=pl.BlockSpec((BB, L, D), lambda i: (i, 0, 0)),
        out_shape=jax.ShapeDtypeStruct((B, L, D), tokens.dtype),
    )(geom, tokens)


# speedup vs baseline: 1.0001x; 1.0001x over previous
import jax
import jax.numpy as jnp
from jax import lax
from jax.experimental import pallas as pl
from jax.experimental.pallas import tpu as pltpu
from jax.experimental.pallas import tpu_sc as plsc

_NC = 2
_NS = 16
_NW = _NC * _NS

_REP = 512


def _tc_add_body(geom_ref, tok_ref, out_ref):
    out_ref[...] = tok_ref[...] + geom_ref[...][:, None, :]


def _tc_select_body(cid_ref, ct_ref, tok_ref, out_ref):
    cid = cid_ref[...]
    ct = ct_ref[...]
    sbit = (cid & 1).astype(jnp.float32)
    vbit = (cid >> 1).astype(jnp.float32)
    a = ct[0][None, :] + sbit * (ct[1] - ct[0])[None, :]
    b = ct[2][None, :] + sbit * (ct[3] - ct[2])[None, :]
    geom = a + vbit * (b - a)
    out_ref[...] = tok_ref[...] + geom[:, None, :]


def _tc_geom_body(obuf_ref, geom_ref, tok_ref, out_ref):
    del obuf_ref
    out_ref[...] = tok_ref[...] + geom_ref[...][:, None, :]


def _make_sc_geom(b_start, b_sc, d, bpw):
    mesh = plsc.VectorSubcoreMesh(
        core_axis_name="c", subcore_axis_name="s",
        num_cores=_NC, num_subcores=_NS)

    def sc_geom(cid, ctable_rep):
        @pl.kernel(
            out_type=jax.ShapeDtypeStruct((b_sc, d), jnp.float32),
            mesh=mesh,
            scratch_types=[
                pltpu.VMEM((bpw,), jnp.int32),
                pltpu.VMEM((bpw, d), jnp.float32),
                pltpu.SemaphoreType.DMA,
            ],
        )
        def run(cid_hbm, ctable_hbm, geom_hbm, c_v, rows_v, sem):
            wid = lax.axis_index("s") * _NC + lax.axis_index("c")
            base = wid * bpw
            pltpu.sync_copy(cid_hbm.at[pl.ds(b_start + base, bpw)], c_v)
            lane = lax.iota(jnp.int32, 16)
            for i in range(bpw // 16):
                s = pl.ds(i * 16, 16)
                rep = (base + i * 16 + lane) & (_REP - 1)
                c_v[s] = c_v[s] + rep * 4
            pltpu.async_copy(ctable_hbm.at[c_v], rows_v, sem).wait()
            pltpu.sync_copy(rows_v, geom_hbm.at[pl.ds(base, bpw)])

        return run(cid, ctable_rep)

    return sc_geom


def kernel(tokens, view_ids, side_ids, view_embed, side_embed):
    B, L, D = tokens.shape
    BB = 128
    NB = B // BB
    NB1 = 4
    B1 = NB1 * BB
    B2 = B - B1
    BPW = B2 // _NW

    vids = view_ids.astype(jnp.int32)
    sids = side_ids.astype(jnp.int32)
    cid = vids * 2 + sids
    cid2d = (vids * 2 + sids).reshape(B, 1)

    ctable_rep = (view_embed[None, :, None, :]
                  + side_embed[None, None, :, :])
    ctable_rep = jnp.broadcast_to(ctable_rep, (_REP, 2, 2, D)).reshape(-1, D)

    geom2 = _make_sc_geom(B1, B2, D, BPW)(cid, ctable_rep)

    obuf = pl.pallas_call(
        _tc_select_body,
        grid=(NB1,),
        in_specs=[
            pl.BlockSpec((BB, 1), lambda i: (i, 0)),
            pl.BlockSpec((8, D), lambda i: (0, 0)),
            pl.BlockSpec((BB, L, D), lambda i: (i, 0, 0)),
        ],
        out_specs=pl.BlockSpec((BB, L, D), lambda i: (i, 0, 0)),
        out_shape=jax.ShapeDtypeStruct((B, L, D), tokens.dtype),
    )(cid2d, ctable_rep, tokens)

    return pl.pallas_call(
        _tc_geom_body,
        grid=(NB - NB1,),
        in_specs=[
            pl.BlockSpec(memory_space=pl.ANY),
            pl.BlockSpec((BB, D), lambda i: (i, 0)),
            pl.BlockSpec((BB, L, D), lambda i: (i + NB1, 0, 0)),
        ],
        out_specs=pl.BlockSpec((BB, L, D), lambda i: (i + NB1, 0, 0)),
        out_shape=jax.ShapeDtypeStruct((B, L, D), tokens.dtype),
        input_output_aliases={0: 0},
    )(obuf, geom2, tokens)


# speedup vs baseline: 1.0013x; 1.0012x over previous
import jax
import jax.numpy as jnp
from jax import lax
from jax.experimental import pallas as pl
from jax.experimental.pallas import tpu as pltpu
from jax.experimental.pallas import tpu_sc as plsc

_NC = 2
_NS = 16
_NW = _NC * _NS

_REP = 512


def _tc_add_body(geom_ref, tok_ref, out_ref):
    out_ref[...] = tok_ref[...] + geom_ref[...][:, None, :]


def _make_sc_geom(b, d, bpw):
    mesh = plsc.VectorSubcoreMesh(
        core_axis_name="c", subcore_axis_name="s",
        num_cores=_NC, num_subcores=_NS)

    def sc_geom(vids, sids, ctable_rep):
        @pl.kernel(
            out_type=jax.ShapeDtypeStruct((b, d), jnp.float32),
            mesh=mesh,
            scratch_types=[
                pltpu.VMEM((bpw,), jnp.int32),
                pltpu.VMEM((bpw,), jnp.int32),
                pltpu.VMEM((bpw, d), jnp.float32),
                pltpu.SemaphoreType.DMA,
            ],
        )
        def run(vids_hbm, sids_hbm, ctable_hbm, geom_hbm,
                v_v, s_v, rows_v, sem):
            wid = lax.axis_index("s") * _NC + lax.axis_index("c")
            base = wid * bpw
            pltpu.sync_copy(vids_hbm.at[pl.ds(base, bpw)], v_v)
            pltpu.sync_copy(sids_hbm.at[pl.ds(base, bpw)], s_v)
            lane = lax.iota(jnp.int32, 16)
            for i in range(bpw // 16):
                s = pl.ds(i * 16, 16)
                rep = (base + i * 16 + lane) & (_REP - 1)
                v_v[s] = v_v[s] * 2 + s_v[s] + rep * 4
            pltpu.async_copy(ctable_hbm.at[v_v], rows_v, sem).wait()
            pltpu.sync_copy(rows_v, geom_hbm.at[pl.ds(base, bpw)])

        return run(vids, sids, ctable_rep)

    return sc_geom


def kernel(tokens, view_ids, side_ids, view_embed, side_embed):
    B, L, D = tokens.shape
    BB = 128
    NB = B // BB
    BPW = B // _NW

    ctable_rep = (view_embed[None, :, None, :]
                  + side_embed[None, None, :, :])
    ctable_rep = jnp.broadcast_to(ctable_rep, (_REP, 2, 2, D)).reshape(-1, D)

    geom = _make_sc_geom(B, D, BPW)(view_ids.astype(jnp.int32),
                                    side_ids.astype(jnp.int32), ctable_rep)

    return pl.pallas_call(
        _tc_add_body,
        grid=(NB,),
        in_specs=[
            pl.BlockSpec((BB, D), lambda i: (i, 0)),
            pl.BlockSpec((BB, L, D), lambda i: (i, 0, 0)),
        ],
        out_specs=pl.BlockSpec((BB, L, D), lambda i: (i, 0, 0)),
        out_shape=jax.ShapeDtypeStruct((B, L, D), tokens.dtype),
    )(geom, tokens)
